# baseline (speedup 1.0000x reference)
import math

import jax
import jax.numpy as jnp
from jax import lax
from jax.experimental import pallas as pl
from jax.experimental.pallas import tpu as pltpu
from jax.experimental.pallas import tpu_sc as plsc

N_PTS = 1048576
N_LEVELS = 12
FPL = 2
LOG2_T = 20
T = 1 << LOG2_T
BASE_RES = 16
HIDDEN = 16
IN_MLP = N_LEVELS * FPL
FIRST_OMEGA = 300.0
PRIME1 = 2654435761

RES = [int(math.floor(BASE_RES * (2.0 ** l))) for l in range(N_LEVELS)]
DENSE = [(r + 1) * (r + 1) <= T for r in RES]

NC, NS = 2, 16
NW = NC * NS
B = 128
PTS_B = B // 2
PPW = N_PTS // NW
NBLK = PPW // PTS_B
NG = B // 16
NROW = 4 * N_LEVELS
GL = NROW * B


def _encode_body(xy_hbm, ftab_hbm, eT_hbm,
                 idxc0, idxc1, cb0, cb1, idxb0, idxb1, wb0, wb1,
                 rowb0, rowb1, outb0, outb1,
                 semg0, semg1, semo0, semo1, semc):
    wid = lax.axis_index("s") * NC + lax.axis_index("c")
    iota16 = lax.iota(jnp.int32, 16)
    half = iota16 >> 1
    parT = (iota16 & 1) * T
    wbase = wid * PPW

    def coords(blk, idxc, cb):
        base = wbase + blk * PTS_B

        def p0(g, c):
            p = base + 8 * g + half
            idxc[pl.ds(g * 16, 16)] = p
            idxc[pl.ds(B + g * 16, 16)] = p + N_PTS
            return c

        lax.fori_loop(0, NG, p0, 0)
        pltpu.async_copy(xy_hbm.at[idxc], cb, semc).wait()

    def pass1(cb, idxb, wb):
        def p1(g, c):
            off = g * 16
            xs = cb[pl.ds(off, 16)]
            ys = cb[pl.ds(B + off, 16)]
            for l in range(N_LEVELS):
                res = RES[l]
                px = xs * jnp.float32(res)
                py = ys * jnp.float32(res)
                ix = px.astype(jnp.int32)
                iy = py.astype(jnp.int32)
                wb[l, 0, pl.ds(off, 16)] = px - ix.astype(jnp.float32)
                wb[l, 1, pl.ds(off, 16)] = py - iy.astype(jnp.float32)
                x1 = jnp.minimum(ix + 1, res)
                y1 = jnp.minimum(iy + 1, res)
                if DENSE[l]:
                    s = res + 1
                    r00 = ix + iy * s
                    r01 = ix + y1 * s
                    r10 = x1 + iy * s
                    r11 = x1 + y1 * s
                else:
                    m = jnp.uint32(T - 1)
                    xu0 = ix.astype(jnp.uint32)
                    xu1 = x1.astype(jnp.uint32)
                    hy0 = iy.astype(jnp.uint32) * jnp.uint32(PRIME1)
                    hy1 = y1.astype(jnp.uint32) * jnp.uint32(PRIME1)
                    r00 = ((xu0 ^ hy0) & m).astype(jnp.int32)
                    r01 = ((xu0 ^ hy1) & m).astype(jnp.int32)
                    r10 = ((xu1 ^ hy0) & m).astype(jnp.int32)
                    r11 = ((xu1 ^ hy1) & m).astype(jnp.int32)
                ltp = 2 * l * T + parT
                idxb[pl.ds((4 * l + 0) * B + off, 16)] = r00 + ltp
                idxb[pl.ds((4 * l + 1) * B + off, 16)] = r01 + ltp
                idxb[pl.ds((4 * l + 2) * B + off, 16)] = r10 + ltp
                idxb[pl.ds((4 * l + 3) * B + off, 16)] = r11 + ltp
            return c

        lax.fori_loop(0, NG, p1, 0)

    def fire(idxb, rowb, semg):
        pltpu.async_copy(ftab_hbm.at[idxb], rowb, semg)

    def drain(idxb, rowb, semg):
        pltpu.make_async_copy(ftab_hbm.at[idxb], rowb, semg).wait()

    def pass2(blk, wb, rowb, outb, semo, first):
        @pl.when(jnp.logical_not(first))
        def _():
            pltpu.make_async_copy(
                outb, eT_hbm.at[:, pl.ds(0, B)], semo).wait()

        def p2(g, c):
            off = g * 16
            for l in range(N_LEVELS):
                wx = wb[l, 0, pl.ds(off, 16)]
                wy = wb[l, 1, pl.ds(off, 16)]
                ex = 1.0 - wx
                ey = 1.0 - wy
                a = (ex * ey) * rowb[pl.ds((4 * l + 0) * B + off, 16)]
                a = a + (ex * wy) * rowb[pl.ds((4 * l + 1) * B + off, 16)]
                a = a + (wx * ey) * rowb[pl.ds((4 * l + 2) * B + off, 16)]
                a = a + (wx * wy) * rowb[pl.ds((4 * l + 3) * B + off, 16)]
                outb[l, pl.ds(off, 16)] = a
            return c

        lax.fori_loop(0, NG, p2, 0)
        base = wbase + blk * PTS_B
        pltpu.async_copy(outb, eT_hbm.at[:, pl.ds(2 * base, B)], semo)

    coords(0, idxc0, cb0)
    pass1(cb0, idxb0, wb0)
    fire(idxb0, rowb0, semg0)

    def outer(k, carry):
        b0 = 2 * k
        b1 = 2 * k + 1

        coords(b1, idxc1, cb1)
        pass1(cb1, idxb1, wb1)
        fire(idxb1, rowb1, semg1)

        drain(idxb0, rowb0, semg0)
        pass2(b0, wb0, rowb0, outb0, semo0, k == 0)

        @pl.when(k < NBLK // 2 - 1)
        def _():
            coords(b0 + 2, idxc0, cb0)
            pass1(cb0, idxb0, wb0)
            fire(idxb0, rowb0, semg0)

        drain(idxb1, rowb1, semg1)
        pass2(b1, wb1, rowb1, outb1, semo1, k == 0)
        return carry

    lax.fori_loop(0, NBLK // 2, outer, 0)

    pltpu.make_async_copy(outb0, eT_hbm.at[:, pl.ds(0, B)], semo0).wait()
    pltpu.make_async_copy(outb1, eT_hbm.at[:, pl.ds(0, B)], semo1).wait()


_hash_encode = pl.kernel(
    _encode_body,
    out_type=jax.ShapeDtypeStruct((N_LEVELS, 2 * N_PTS), jnp.float32),
    mesh=plsc.VectorSubcoreMesh(core_axis_name="c", subcore_axis_name="s"),
    scratch_types=[
        pltpu.VMEM((2 * B,), jnp.int32),
        pltpu.VMEM((2 * B,), jnp.int32),
        pltpu.VMEM((2 * B,), jnp.float32),
        pltpu.VMEM((2 * B,), jnp.float32),
        pltpu.VMEM((GL,), jnp.int32),
        pltpu.VMEM((GL,), jnp.int32),
        pltpu.VMEM((N_LEVELS, 2, B), jnp.float32),
        pltpu.VMEM((N_LEVELS, 2, B), jnp.float32),
        pltpu.VMEM((GL,), jnp.float32),
        pltpu.VMEM((GL,), jnp.float32),
        pltpu.VMEM((N_LEVELS, B), jnp.float32),
        pltpu.VMEM((N_LEVELS, B), jnp.float32),
        pltpu.SemaphoreType.DMA,
        pltpu.SemaphoreType.DMA,
        pltpu.SemaphoreType.DMA,
        pltpu.SemaphoreType.DMA,
        pltpu.SemaphoreType.DMA,
    ],
)


BT = 4096


def _mlp_body(e_ref, a0, a1, b0, w1, b1, w2, b2, w3, b3, o_ref):
    e = e_ref[...]
    er = jnp.concatenate([e[:, 1:], e[:, :1]], axis=1)
    h = jnp.dot(a0[...], e, preferred_element_type=jnp.float32)
    h = h + jnp.dot(a1[...], er, preferred_element_type=jnp.float32)
    h = jnp.sin(FIRST_OMEGA * (h + b0[...]))
    h = jnp.sin(jnp.dot(w1[...], h, preferred_element_type=jnp.float32) + b1[...])
    h = jnp.sin(jnp.dot(w2[...], h, preferred_element_type=jnp.float32) + b2[...])
    o_ref[...] = jnp.dot(w3[...], h, preferred_element_type=jnp.float32) + b3[...]


def _mlp(eI, A0, A1, b0, W1, b1, W2, b2, W3, b3):
    full = lambda shape: pl.BlockSpec(shape, lambda i: (0, 0))
    return pl.pallas_call(
        _mlp_body,
        grid=(N_PTS // BT,),
        in_specs=[
            pl.BlockSpec((N_LEVELS, 2 * BT), lambda i: (0, i)),
            full((HIDDEN, N_LEVELS)), full((HIDDEN, N_LEVELS)),
            full((HIDDEN, 1)),
            full((HIDDEN, HIDDEN)), full((HIDDEN, 1)),
            full((HIDDEN, HIDDEN)), full((HIDDEN, 1)),
            full((1, HIDDEN)), full((1, 1)),
        ],
        out_specs=pl.BlockSpec((1, 2 * BT), lambda i: (0, i)),
        out_shape=jax.ShapeDtypeStruct((1, 2 * N_PTS), jnp.float32),
    )(eI, A0, A1, b0, W1, b1, W2, b2, W3, b3)


def kernel(input, table, W0, b0, W1, b1, W2, b2, W3, b3):
    xy = input.T.reshape(2 * N_PTS)
    ftab = table.transpose(0, 2, 1).reshape(N_LEVELS * FPL * T)
    eI = _hash_encode(xy, ftab)
    A0 = W0[:, 0::2]
    A1 = W0[:, 1::2]
    out2 = _mlp(eI, A0, A1, b0.reshape(HIDDEN, 1), W1, b1.reshape(HIDDEN, 1),
                W2, b2.reshape(HIDDEN, 1), W3, b3.reshape(1, 1))
    return out2.reshape(2 * N_PTS)[0::2].reshape(N_PTS, 1)


# speedup vs baseline: 2.7242x; 2.7242x over previous
import math

import jax
import jax.numpy as jnp
from jax import lax
from jax.experimental import pallas as pl
from jax.experimental.pallas import tpu as pltpu
from jax.experimental.pallas import tpu_sc as plsc

N_PTS = 1048576
N_LEVELS = 12
FPL = 2
LOG2_T = 20
T = 1 << LOG2_T
BASE_RES = 16
HIDDEN = 16
IN_MLP = N_LEVELS * FPL
FIRST_OMEGA = 300.0
PRIME1 = 2654435761

RES = [int(math.floor(BASE_RES * (2.0 ** l))) for l in range(N_LEVELS)]
DENSE = [(r + 1) * (r + 1) <= T for r in RES]

KD = 4
SD = [(RES[l] + 1) * (RES[l] + 1) for l in range(KD)]
SP = [((s + 7) // 8) * 8 for s in SD]
TOFF = [0] * KD
for _l in range(1, KD):
    TOFF[_l] = TOFF[_l - 1] + 2 * SP[_l - 1]
TS = TOFF[KD - 1] + 2 * SP[KD - 1]

NC, NS = 2, 16
NW = NC * NS
B = 128
PTS_B = B // 2
PPW = N_PTS // NW
NBLK = PPW // PTS_B
NG = B // 16
NSL = N_LEVELS - KD
NROW = 4 * NSL
GL = NROW * B


def _encode_body(xy_hbm, ftab_hbm, eT_hbm,
                 idxc0, idxc1, cb0, cb1, idxb0, idxb1, wb0, wb1,
                 rowb0, rowb1, outb0, outb1, tabs,
                 semg0, semg1, semo0, semo1, semc):
    wid = lax.axis_index("s") * NC + lax.axis_index("c")
    iota16 = lax.iota(jnp.int32, 16)
    half = iota16 >> 1
    par = iota16 & 1
    parT = par * T
    wbase = wid * PPW

    tcp = []
    for l in range(KD):
        for p in range(FPL):
            tcp.append(pltpu.async_copy(
                ftab_hbm.at[pl.ds((2 * l + p) * T, SP[l])],
                tabs.at[pl.ds(TOFF[l] + p * SP[l], SP[l])], semc))
    for c in tcp:
        c.wait()

    def coords(blk, idxc, cb):
        base = wbase + blk * PTS_B

        def p0(g, c):
            p = base + 8 * g + half
            idxc[pl.ds(g * 16, 16)] = p
            idxc[pl.ds(B + g * 16, 16)] = p + N_PTS
            return c

        lax.fori_loop(0, NG, p0, 0)
        pltpu.async_copy(xy_hbm.at[idxc], cb, semc).wait()

    def pass1(cb, idxb, wb):
        def p1(g, c):
            off = g * 16
            xs = cb[pl.ds(off, 16)]
            ys = cb[pl.ds(B + off, 16)]
            for l in range(KD, N_LEVELS):
                res = RES[l]
                px = xs * jnp.float32(res)
                py = ys * jnp.float32(res)
                ix = px.astype(jnp.int32)
                iy = py.astype(jnp.int32)
                sl = l - KD
                wb[sl, 0, pl.ds(off, 16)] = px - ix.astype(jnp.float32)
                wb[sl, 1, pl.ds(off, 16)] = py - iy.astype(jnp.float32)
                x1 = jnp.minimum(ix + 1, res)
                y1 = jnp.minimum(iy + 1, res)
                if DENSE[l]:
                    s = res + 1
                    r00 = ix + iy * s
                    r01 = ix + y1 * s
                    r10 = x1 + iy * s
                    r11 = x1 + y1 * s
                else:
                    m = jnp.uint32(T - 1)
                    xu0 = ix.astype(jnp.uint32)
                    xu1 = x1.astype(jnp.uint32)
                    hy0 = iy.astype(jnp.uint32) * jnp.uint32(PRIME1)
                    hy1 = y1.astype(jnp.uint32) * jnp.uint32(PRIME1)
                    r00 = ((xu0 ^ hy0) & m).astype(jnp.int32)
                    r01 = ((xu0 ^ hy1) & m).astype(jnp.int32)
                    r10 = ((xu1 ^ hy0) & m).astype(jnp.int32)
                    r11 = ((xu1 ^ hy1) & m).astype(jnp.int32)
                ltp = 2 * l * T + parT
                idxb[pl.ds((4 * sl + 0) * B + off, 16)] = r00 + ltp
                idxb[pl.ds((4 * sl + 1) * B + off, 16)] = r01 + ltp
                idxb[pl.ds((4 * sl + 2) * B + off, 16)] = r10 + ltp
                idxb[pl.ds((4 * sl + 3) * B + off, 16)] = r11 + ltp
            return c

        lax.fori_loop(0, NG, p1, 0)

    def fire(idxb, rowb, semg):
        pltpu.async_copy(ftab_hbm.at[idxb], rowb, semg)

    def drain(idxb, rowb, semg):
        pltpu.make_async_copy(ftab_hbm.at[idxb], rowb, semg).wait()

    def pass2(blk, cb, wb, rowb, outb, semo, first):
        @pl.when(jnp.logical_not(first))
        def _():
            pltpu.make_async_copy(
                outb, eT_hbm.at[:, pl.ds(0, B)], semo).wait()

        def p2(g, c):
            off = g * 16
            xs = cb[pl.ds(off, 16)]
            ys = cb[pl.ds(B + off, 16)]
            for l in range(KD):
                res = RES[l]
                px = xs * jnp.float32(res)
                py = ys * jnp.float32(res)
                ix = px.astype(jnp.int32)
                iy = py.astype(jnp.int32)
                wx = px - ix.astype(jnp.float32)
                wy = py - iy.astype(jnp.float32)
                ex = 1.0 - wx
                ey = 1.0 - wy
                x1 = jnp.minimum(ix + 1, res)
                y1 = jnp.minimum(iy + 1, res)
                s = res + 1
                lp = TOFF[l] + par * SP[l]
                b00 = ix + iy * s + lp
                b01 = ix + y1 * s + lp
                b10 = x1 + iy * s + lp
                b11 = x1 + y1 * s + lp
                a = (ex * ey) * plsc.load_gather(tabs, [b00])
                a = a + (ex * wy) * plsc.load_gather(tabs, [b01])
                a = a + (wx * ey) * plsc.load_gather(tabs, [b10])
                a = a + (wx * wy) * plsc.load_gather(tabs, [b11])
                outb[l, pl.ds(off, 16)] = a
            for l in range(KD, N_LEVELS):
                sl = l - KD
                wx = wb[sl, 0, pl.ds(off, 16)]
                wy = wb[sl, 1, pl.ds(off, 16)]
                ex = 1.0 - wx
                ey = 1.0 - wy
                a = (ex * ey) * rowb[pl.ds((4 * sl + 0) * B + off, 16)]
                a = a + (ex * wy) * rowb[pl.ds((4 * sl + 1) * B + off, 16)]
                a = a + (wx * ey) * rowb[pl.ds((4 * sl + 2) * B + off, 16)]
                a = a + (wx * wy) * rowb[pl.ds((4 * sl + 3) * B + off, 16)]
                outb[l, pl.ds(off, 16)] = a
            return c

        lax.fori_loop(0, NG, p2, 0)
        base = wbase + blk * PTS_B
        pltpu.async_copy(outb, eT_hbm.at[:, pl.ds(2 * base, B)], semo)

    coords(0, idxc0, cb0)
    pass1(cb0, idxb0, wb0)
    fire(idxb0, rowb0, semg0)

    def outer(k, carry):
        b0 = 2 * k
        b1 = 2 * k + 1

        coords(b1, idxc1, cb1)
        pass1(cb1, idxb1, wb1)
        fire(idxb1, rowb1, semg1)

        drain(idxb0, rowb0, semg0)
        pass2(b0, cb0, wb0, rowb0, outb0, semo0, k == 0)

        @pl.when(k < NBLK // 2 - 1)
        def _():
            coords(b0 + 2, idxc0, cb0)
            pass1(cb0, idxb0, wb0)
            fire(idxb0, rowb0, semg0)

        drain(idxb1, rowb1, semg1)
        pass2(b1, cb1, wb1, rowb1, outb1, semo1, k == 0)
        return carry

    lax.fori_loop(0, NBLK // 2, outer, 0)

    pltpu.make_async_copy(outb0, eT_hbm.at[:, pl.ds(0, B)], semo0).wait()
    pltpu.make_async_copy(outb1, eT_hbm.at[:, pl.ds(0, B)], semo1).wait()


_hash_encode = pl.kernel(
    _encode_body,
    out_type=jax.ShapeDtypeStruct((N_LEVELS, 2 * N_PTS), jnp.float32),
    mesh=plsc.VectorSubcoreMesh(core_axis_name="c", subcore_axis_name="s"),
    compiler_params=pltpu.CompilerParams(needs_layout_passes=False),
    scratch_types=[
        pltpu.VMEM((2 * B,), jnp.int32),
        pltpu.VMEM((2 * B,), jnp.int32),
        pltpu.VMEM((2 * B,), jnp.float32),
        pltpu.VMEM((2 * B,), jnp.float32),
        pltpu.VMEM((GL,), jnp.int32),
        pltpu.VMEM((GL,), jnp.int32),
        pltpu.VMEM((NSL, 2, B), jnp.float32),
        pltpu.VMEM((NSL, 2, B), jnp.float32),
        pltpu.VMEM((GL,), jnp.float32),
        pltpu.VMEM((GL,), jnp.float32),
        pltpu.VMEM((N_LEVELS, B), jnp.float32),
        pltpu.VMEM((N_LEVELS, B), jnp.float32),
        pltpu.VMEM((TS,), jnp.float32),
        pltpu.SemaphoreType.DMA,
        pltpu.SemaphoreType.DMA,
        pltpu.SemaphoreType.DMA,
        pltpu.SemaphoreType.DMA,
        pltpu.SemaphoreType.DMA,
    ],
)


BT = 4096


def _mlp_body(e_ref, a0, a1, b0, w1, b1, w2, b2, w3, b3, o_ref):
    e = e_ref[...]
    er = jnp.concatenate([e[:, 1:], e[:, :1]], axis=1)
    h = jnp.dot(a0[...], e, preferred_element_type=jnp.float32)
    h = h + jnp.dot(a1[...], er, preferred_element_type=jnp.float32)
    h = jnp.sin(FIRST_OMEGA * (h + b0[...]))
    h = jnp.sin(jnp.dot(w1[...], h, preferred_element_type=jnp.float32) + b1[...])
    h = jnp.sin(jnp.dot(w2[...], h, preferred_element_type=jnp.float32) + b2[...])
    o_ref[...] = jnp.dot(w3[...], h, preferred_element_type=jnp.float32) + b3[...]


def _mlp(eI, A0, A1, b0, W1, b1, W2, b2, W3, b3):
    full = lambda shape: pl.BlockSpec(shape, lambda i: (0, 0))
    return pl.pallas_call(
        _mlp_body,
        grid=(N_PTS // BT,),
        in_specs=[
            pl.BlockSpec((N_LEVELS, 2 * BT), lambda i: (0, i)),
            full((HIDDEN, N_LEVELS)), full((HIDDEN, N_LEVELS)),
            full((HIDDEN, 1)),
            full((HIDDEN, HIDDEN)), full((HIDDEN, 1)),
            full((HIDDEN, HIDDEN)), full((HIDDEN, 1)),
            full((1, HIDDEN)), full((1, 1)),
        ],
        out_specs=pl.BlockSpec((1, 2 * BT), lambda i: (0, i)),
        out_shape=jax.ShapeDtypeStruct((1, 2 * N_PTS), jnp.float32),
    )(eI, A0, A1, b0, W1, b1, W2, b2, W3, b3)


def kernel(input, table, W0, b0, W1, b1, W2, b2, W3, b3):
    xy = input.T.reshape(2 * N_PTS)
    ftab = table.transpose(0, 2, 1).reshape(N_LEVELS * FPL * T)
    eI = _hash_encode(xy, ftab)
    A0 = W0[:, 0::2]
    A1 = W0[:, 1::2]
    out2 = _mlp(eI, A0, A1, b0.reshape(HIDDEN, 1), W1, b1.reshape(HIDDEN, 1),
                W2, b2.reshape(HIDDEN, 1), W3, b3.reshape(1, 1))
    return out2.reshape(2 * N_PTS)[0::2].reshape(N_PTS, 1)


# speedup vs baseline: 3.2679x; 1.1996x over previous
import math

import jax
import jax.numpy as jnp
from jax import lax
from jax.experimental import pallas as pl
from jax.experimental.pallas import tpu as pltpu
from jax.experimental.pallas import tpu_sc as plsc

N_PTS = 1048576
N_LEVELS = 12
FPL = 2
LOG2_T = 20
T = 1 << LOG2_T
BASE_RES = 16
HIDDEN = 16
IN_MLP = N_LEVELS * FPL
FIRST_OMEGA = 300.0
PRIME1 = 2654435761

RES = [int(math.floor(BASE_RES * (2.0 ** l))) for l in range(N_LEVELS)]
DENSE = [(r + 1) * (r + 1) <= T for r in RES]

KD = 4
SD = [(RES[l] + 1) * (RES[l] + 1) for l in range(KD)]
SP = [((s + 7) // 8) * 8 for s in SD]
TOFF = [0] * KD
for _l in range(1, KD):
    TOFF[_l] = TOFF[_l - 1] + 2 * SP[_l - 1]
TS = TOFF[KD - 1] + 2 * SP[KD - 1]

NC, NS = 2, 16
NW = NC * NS
B = 128
PTS_B = B // 2
NG = B // 16
NSL = N_LEVELS - KD
NROW = 4 * NSL
GL = NROW * B

NCHUNK = 2
CPTS = N_PTS // NCHUNK
PPW = CPTS // NW
NBLK = PPW // PTS_B


def _make_encode(chunk_off):
    def _encode_body(xy_hbm, ftab_hbm, eT_hbm,
                     idxc0, idxc1, cb0, cb1, idxb0, idxb1, wb0, wb1,
                     rowb0, rowb1, outb0, outb1, tabs,
                     semg0, semg1, semo0, semo1, semc):
        wid = lax.axis_index("s") * NC + lax.axis_index("c")
        iota16 = lax.iota(jnp.int32, 16)
        half = iota16 >> 1
        par = iota16 & 1
        parT = par * T
        wbase = wid * PPW

        tcp = []
        for l in range(KD):
            for p in range(FPL):
                tcp.append(pltpu.async_copy(
                    ftab_hbm.at[pl.ds((2 * l + p) * T, SP[l])],
                    tabs.at[pl.ds(TOFF[l] + p * SP[l], SP[l])], semc))
        for c in tcp:
            c.wait()

        def coords(blk, idxc, cb):
            base = chunk_off + wbase + blk * PTS_B

            def p0(g, c):
                p = base + 8 * g + half
                idxc[pl.ds(g * 16, 16)] = p
                idxc[pl.ds(B + g * 16, 16)] = p + N_PTS
                return c

            lax.fori_loop(0, NG, p0, 0)
            pltpu.async_copy(xy_hbm.at[idxc], cb, semc).wait()

        def pass1(cb, idxb, wb):
            def p1(g, c):
                off = g * 16
                xs = cb[pl.ds(off, 16)]
                ys = cb[pl.ds(B + off, 16)]
                for l in range(KD, N_LEVELS):
                    res = RES[l]
                    px = xs * jnp.float32(res)
                    py = ys * jnp.float32(res)
                    ix = px.astype(jnp.int32)
                    iy = py.astype(jnp.int32)
                    sl = l - KD
                    wb[sl, 0, pl.ds(off, 16)] = px - ix.astype(jnp.float32)
                    wb[sl, 1, pl.ds(off, 16)] = py - iy.astype(jnp.float32)
                    x1 = jnp.minimum(ix + 1, res)
                    y1 = jnp.minimum(iy + 1, res)
                    if DENSE[l]:
                        s = res + 1
                        r00 = ix + iy * s
                        r01 = ix + y1 * s
                        r10 = x1 + iy * s
                        r11 = x1 + y1 * s
                    else:
                        m = jnp.uint32(T - 1)
                        xu0 = ix.astype(jnp.uint32)
                        xu1 = x1.astype(jnp.uint32)
                        hy0 = iy.astype(jnp.uint32) * jnp.uint32(PRIME1)
                        hy1 = y1.astype(jnp.uint32) * jnp.uint32(PRIME1)
                        r00 = ((xu0 ^ hy0) & m).astype(jnp.int32)
                        r01 = ((xu0 ^ hy1) & m).astype(jnp.int32)
                        r10 = ((xu1 ^ hy0) & m).astype(jnp.int32)
                        r11 = ((xu1 ^ hy1) & m).astype(jnp.int32)
                    ltp = 2 * l * T + parT
                    idxb[pl.ds((4 * sl + 0) * B + off, 16)] = r00 + ltp
                    idxb[pl.ds((4 * sl + 1) * B + off, 16)] = r01 + ltp
                    idxb[pl.ds((4 * sl + 2) * B + off, 16)] = r10 + ltp
                    idxb[pl.ds((4 * sl + 3) * B + off, 16)] = r11 + ltp
                return c

            lax.fori_loop(0, NG, p1, 0)

        def fire(idxb, rowb, semg):
            pltpu.async_copy(ftab_hbm.at[idxb], rowb, semg)

        def drain(idxb, rowb, semg):
            pltpu.make_async_copy(ftab_hbm.at[idxb], rowb, semg).wait()

        def pass2(blk, cb, wb, rowb, outb, semo, first):
            @pl.when(jnp.logical_not(first))
            def _():
                pltpu.make_async_copy(
                    outb, eT_hbm.at[:, pl.ds(0, B)], semo).wait()

            def p2(g, c):
                off = g * 16
                xs = cb[pl.ds(off, 16)]
                ys = cb[pl.ds(B + off, 16)]
                for l in range(KD):
                    res = RES[l]
                    px = xs * jnp.float32(res)
                    py = ys * jnp.float32(res)
                    ix = px.astype(jnp.int32)
                    iy = py.astype(jnp.int32)
                    wx = px - ix.astype(jnp.float32)
                    wy = py - iy.astype(jnp.float32)
                    ex = 1.0 - wx
                    ey = 1.0 - wy
                    x1 = jnp.minimum(ix + 1, res)
                    y1 = jnp.minimum(iy + 1, res)
                    s = res + 1
                    lp = TOFF[l] + par * SP[l]
                    b00 = ix + iy * s + lp
                    b01 = ix + y1 * s + lp
                    b10 = x1 + iy * s + lp
                    b11 = x1 + y1 * s + lp
                    a = (ex * ey) * plsc.load_gather(tabs, [b00])
                    a = a + (ex * wy) * plsc.load_gather(tabs, [b01])
                    a = a + (wx * ey) * plsc.load_gather(tabs, [b10])
                    a = a + (wx * wy) * plsc.load_gather(tabs, [b11])
                    outb[l, pl.ds(off, 16)] = a
                for l in range(KD, N_LEVELS):
                    sl = l - KD
                    wx = wb[sl, 0, pl.ds(off, 16)]
                    wy = wb[sl, 1, pl.ds(off, 16)]
                    ex = 1.0 - wx
                    ey = 1.0 - wy
                    a = (ex * ey) * rowb[pl.ds((4 * sl + 0) * B + off, 16)]
                    a = a + (ex * wy) * rowb[pl.ds((4 * sl + 1) * B + off, 16)]
                    a = a + (wx * ey) * rowb[pl.ds((4 * sl + 2) * B + off, 16)]
                    a = a + (wx * wy) * rowb[pl.ds((4 * sl + 3) * B + off, 16)]
                    outb[l, pl.ds(off, 16)] = a
                return c

            lax.fori_loop(0, NG, p2, 0)
            base = wbase + blk * PTS_B
            pltpu.async_copy(outb, eT_hbm.at[:, pl.ds(2 * base, B)], semo)

        coords(0, idxc0, cb0)
        pass1(cb0, idxb0, wb0)
        fire(idxb0, rowb0, semg0)

        def outer(k, carry):
            b0 = 2 * k
            b1 = 2 * k + 1

            coords(b1, idxc1, cb1)
            pass1(cb1, idxb1, wb1)
            fire(idxb1, rowb1, semg1)

            drain(idxb0, rowb0, semg0)
            pass2(b0, cb0, wb0, rowb0, outb0, semo0, k == 0)

            @pl.when(k < NBLK // 2 - 1)
            def _():
                coords(b0 + 2, idxc0, cb0)
                pass1(cb0, idxb0, wb0)
                fire(idxb0, rowb0, semg0)

            drain(idxb1, rowb1, semg1)
            pass2(b1, cb1, wb1, rowb1, outb1, semo1, k == 0)
            return carry

        lax.fori_loop(0, NBLK // 2, outer, 0)

        pltpu.make_async_copy(outb0, eT_hbm.at[:, pl.ds(0, B)], semo0).wait()
        pltpu.make_async_copy(outb1, eT_hbm.at[:, pl.ds(0, B)], semo1).wait()

    return pl.kernel(
        _encode_body,
        out_type=jax.ShapeDtypeStruct((N_LEVELS, 2 * CPTS), jnp.float32),
        mesh=plsc.VectorSubcoreMesh(core_axis_name="c", subcore_axis_name="s"),
        compiler_params=pltpu.CompilerParams(needs_layout_passes=False),
        scratch_types=[
            pltpu.VMEM((2 * B,), jnp.int32),
            pltpu.VMEM((2 * B,), jnp.int32),
            pltpu.VMEM((2 * B,), jnp.float32),
            pltpu.VMEM((2 * B,), jnp.float32),
            pltpu.VMEM((GL,), jnp.int32),
            pltpu.VMEM((GL,), jnp.int32),
            pltpu.VMEM((NSL, 2, B), jnp.float32),
            pltpu.VMEM((NSL, 2, B), jnp.float32),
            pltpu.VMEM((GL,), jnp.float32),
            pltpu.VMEM((GL,), jnp.float32),
            pltpu.VMEM((N_LEVELS, B), jnp.float32),
            pltpu.VMEM((N_LEVELS, B), jnp.float32),
            pltpu.VMEM((TS,), jnp.float32),
            pltpu.SemaphoreType.DMA,
            pltpu.SemaphoreType.DMA,
            pltpu.SemaphoreType.DMA,
            pltpu.SemaphoreType.DMA,
            pltpu.SemaphoreType.DMA,
        ],
    )


_hash_encode = [_make_encode(h * CPTS) for h in range(NCHUNK)]


BT = 4096


def _mlp_body(e_ref, a0, a1, b0, w1, b1, w2, b2, w3, b3, o_ref):
    e = e_ref[...]
    er = jnp.concatenate([e[:, 1:], e[:, :1]], axis=1)
    h = jnp.dot(a0[...], e, preferred_element_type=jnp.float32)
    h = h + jnp.dot(a1[...], er, preferred_element_type=jnp.float32)
    h = jnp.sin(FIRST_OMEGA * (h + b0[...]))
    h = jnp.sin(jnp.dot(w1[...], h, preferred_element_type=jnp.float32) + b1[...])
    h = jnp.sin(jnp.dot(w2[...], h, preferred_element_type=jnp.float32) + b2[...])
    o_ref[...] = jnp.dot(w3[...], h, preferred_element_type=jnp.float32) + b3[...]


def _mlp(eI, A0, A1, b0, W1, b1, W2, b2, W3, b3):
    full = lambda shape: pl.BlockSpec(shape, lambda i: (0, 0))
    return pl.pallas_call(
        _mlp_body,
        grid=(CPTS // BT,),
        in_specs=[
            pl.BlockSpec((N_LEVELS, 2 * BT), lambda i: (0, i)),
            full((HIDDEN, N_LEVELS)), full((HIDDEN, N_LEVELS)),
            full((HIDDEN, 1)),
            full((HIDDEN, HIDDEN)), full((HIDDEN, 1)),
            full((HIDDEN, HIDDEN)), full((HIDDEN, 1)),
            full((1, HIDDEN)), full((1, 1)),
        ],
        out_specs=pl.BlockSpec((1, 2 * BT), lambda i: (0, i)),
        out_shape=jax.ShapeDtypeStruct((1, 2 * CPTS), jnp.float32),
    )(eI, A0, A1, b0, W1, b1, W2, b2, W3, b3)


def kernel(input, table, W0, b0, W1, b1, W2, b2, W3, b3):
    xy = input.T.reshape(2 * N_PTS)
    ftab = table.transpose(0, 2, 1).reshape(N_LEVELS * FPL * T)
    A0 = W0[:, 0::2]
    A1 = W0[:, 1::2]
    mlp_args = (A0, A1, b0.reshape(HIDDEN, 1), W1, b1.reshape(HIDDEN, 1),
                W2, b2.reshape(HIDDEN, 1), W3, b3.reshape(1, 1))
    outs = []
    for h in range(NCHUNK):
        eI = _hash_encode[h](xy, ftab)
        outs.append(_mlp(eI, *mlp_args))
    return jnp.concatenate(
        [o.reshape(2 * CPTS)[0::2] for o in outs]).reshape(N_PTS, 1)


# speedup vs baseline: 3.5832x; 1.0965x over previous
import math

import jax
import jax.numpy as jnp
from jax import lax
from jax.experimental import pallas as pl
from jax.experimental.pallas import tpu as pltpu
from jax.experimental.pallas import tpu_sc as plsc

N_PTS = 1048576
N_LEVELS = 12
FPL = 2
LOG2_T = 20
T = 1 << LOG2_T
BASE_RES = 16
HIDDEN = 16
IN_MLP = N_LEVELS * FPL
FIRST_OMEGA = 300.0
PRIME1 = 2654435761

RES = [int(math.floor(BASE_RES * (2.0 ** l))) for l in range(N_LEVELS)]
DENSE = [(r + 1) * (r + 1) <= T for r in RES]

KD = 4
SD = [(RES[l] + 1) * (RES[l] + 1) for l in range(KD)]
SP = [((s + 7) // 8) * 8 for s in SD]
TOFF = [0] * KD
for _l in range(1, KD):
    TOFF[_l] = TOFF[_l - 1] + 2 * SP[_l - 1]
TS = TOFF[KD - 1] + 2 * SP[KD - 1]

NC, NS = 2, 16
NW = NC * NS
B = 128
PTS_B = B // 2
NG = B // 16
NSL = N_LEVELS - KD
NROW = 4 * NSL
GL = NROW * B

NCHUNK = 4
CPTS = N_PTS // NCHUNK
PPW = CPTS // NW
NBLK = PPW // PTS_B


def _make_encode(chunk_off):
    def _encode_body(xy_hbm, ftab_hbm, eT_hbm,
                     idxc0, idxc1, cb0, cb1, idxb0, idxb1, wb0, wb1,
                     rowb0, rowb1, outb0, outb1, tabs,
                     semg0, semg1, semo0, semo1, semc):
        wid = lax.axis_index("s") * NC + lax.axis_index("c")
        iota16 = lax.iota(jnp.int32, 16)
        half = iota16 >> 1
        par = iota16 & 1
        parT = par * T
        wbase = wid * PPW

        tcp = []
        for l in range(KD):
            for p in range(FPL):
                tcp.append(pltpu.async_copy(
                    ftab_hbm.at[pl.ds((2 * l + p) * T, SP[l])],
                    tabs.at[pl.ds(TOFF[l] + p * SP[l], SP[l])], semc))
        for c in tcp:
            c.wait()

        def coords(blk, idxc, cb):
            base = chunk_off + wbase + blk * PTS_B

            def p0(g, c):
                p = base + 8 * g + half
                idxc[pl.ds(g * 16, 16)] = p
                idxc[pl.ds(B + g * 16, 16)] = p + N_PTS
                return c

            lax.fori_loop(0, NG, p0, 0)
            pltpu.async_copy(xy_hbm.at[idxc], cb, semc).wait()

        def pass1(cb, idxb, wb):
            def p1(g, c):
                off = g * 16
                xs = cb[pl.ds(off, 16)]
                ys = cb[pl.ds(B + off, 16)]
                for l in range(KD, N_LEVELS):
                    res = RES[l]
                    px = xs * jnp.float32(res)
                    py = ys * jnp.float32(res)
                    ix = px.astype(jnp.int32)
                    iy = py.astype(jnp.int32)
                    sl = l - KD
                    wb[sl, 0, pl.ds(off, 16)] = px - ix.astype(jnp.float32)
                    wb[sl, 1, pl.ds(off, 16)] = py - iy.astype(jnp.float32)
                    x1 = jnp.minimum(ix + 1, res)
                    y1 = jnp.minimum(iy + 1, res)
                    if DENSE[l]:
                        s = res + 1
                        r00 = ix + iy * s
                        r01 = ix + y1 * s
                        r10 = x1 + iy * s
                        r11 = x1 + y1 * s
                    else:
                        m = jnp.uint32(T - 1)
                        xu0 = ix.astype(jnp.uint32)
                        xu1 = x1.astype(jnp.uint32)
                        hy0 = iy.astype(jnp.uint32) * jnp.uint32(PRIME1)
                        hy1 = y1.astype(jnp.uint32) * jnp.uint32(PRIME1)
                        r00 = ((xu0 ^ hy0) & m).astype(jnp.int32)
                        r01 = ((xu0 ^ hy1) & m).astype(jnp.int32)
                        r10 = ((xu1 ^ hy0) & m).astype(jnp.int32)
                        r11 = ((xu1 ^ hy1) & m).astype(jnp.int32)
                    ltp = 2 * l * T + parT
                    idxb[pl.ds((4 * sl + 0) * B + off, 16)] = r00 + ltp
                    idxb[pl.ds((4 * sl + 1) * B + off, 16)] = r01 + ltp
                    idxb[pl.ds((4 * sl + 2) * B + off, 16)] = r10 + ltp
                    idxb[pl.ds((4 * sl + 3) * B + off, 16)] = r11 + ltp
                return c

            lax.fori_loop(0, NG, p1, 0)

        def fire(idxb, rowb, semg):
            pltpu.async_copy(ftab_hbm.at[idxb], rowb, semg)

        def drain(idxb, rowb, semg):
            pltpu.make_async_copy(ftab_hbm.at[idxb], rowb, semg).wait()

        def pass2(blk, cb, wb, rowb, outb, semo, first):
            @pl.when(jnp.logical_not(first))
            def _():
                pltpu.make_async_copy(
                    outb, eT_hbm.at[:, pl.ds(0, B)], semo).wait()

            def p2(g, c):
                off = g * 16
                xs = cb[pl.ds(off, 16)]
                ys = cb[pl.ds(B + off, 16)]
                for l in range(KD):
                    res = RES[l]
                    px = xs * jnp.float32(res)
                    py = ys * jnp.float32(res)
                    ix = px.astype(jnp.int32)
                    iy = py.astype(jnp.int32)
                    wx = px - ix.astype(jnp.float32)
                    wy = py - iy.astype(jnp.float32)
                    ex = 1.0 - wx
                    ey = 1.0 - wy
                    x1 = jnp.minimum(ix + 1, res)
                    y1 = jnp.minimum(iy + 1, res)
                    s = res + 1
                    lp = TOFF[l] + par * SP[l]
                    b00 = ix + iy * s + lp
                    b01 = ix + y1 * s + lp
                    b10 = x1 + iy * s + lp
                    b11 = x1 + y1 * s + lp
                    a = (ex * ey) * plsc.load_gather(tabs, [b00])
                    a = a + (ex * wy) * plsc.load_gather(tabs, [b01])
                    a = a + (wx * ey) * plsc.load_gather(tabs, [b10])
                    a = a + (wx * wy) * plsc.load_gather(tabs, [b11])
                    outb[l, pl.ds(off, 16)] = a
                for l in range(KD, N_LEVELS):
                    sl = l - KD
                    wx = wb[sl, 0, pl.ds(off, 16)]
                    wy = wb[sl, 1, pl.ds(off, 16)]
                    ex = 1.0 - wx
                    ey = 1.0 - wy
                    a = (ex * ey) * rowb[pl.ds((4 * sl + 0) * B + off, 16)]
                    a = a + (ex * wy) * rowb[pl.ds((4 * sl + 1) * B + off, 16)]
                    a = a + (wx * ey) * rowb[pl.ds((4 * sl + 2) * B + off, 16)]
                    a = a + (wx * wy) * rowb[pl.ds((4 * sl + 3) * B + off, 16)]
                    outb[l, pl.ds(off, 16)] = a
                return c

            lax.fori_loop(0, NG, p2, 0)
            base = wbase + blk * PTS_B
            pltpu.async_copy(outb, eT_hbm.at[:, pl.ds(2 * base, B)], semo)

        coords(0, idxc0, cb0)
        pass1(cb0, idxb0, wb0)
        fire(idxb0, rowb0, semg0)

        def outer(k, carry):
            b0 = 2 * k
            b1 = 2 * k + 1

            coords(b1, idxc1, cb1)
            pass1(cb1, idxb1, wb1)
            fire(idxb1, rowb1, semg1)

            drain(idxb0, rowb0, semg0)
            pass2(b0, cb0, wb0, rowb0, outb0, semo0, k == 0)

            @pl.when(k < NBLK // 2 - 1)
            def _():
                coords(b0 + 2, idxc0, cb0)
                pass1(cb0, idxb0, wb0)
                fire(idxb0, rowb0, semg0)

            drain(idxb1, rowb1, semg1)
            pass2(b1, cb1, wb1, rowb1, outb1, semo1, k == 0)
            return carry

        lax.fori_loop(0, NBLK // 2, outer, 0)

        pltpu.make_async_copy(outb0, eT_hbm.at[:, pl.ds(0, B)], semo0).wait()
        pltpu.make_async_copy(outb1, eT_hbm.at[:, pl.ds(0, B)], semo1).wait()

    return pl.kernel(
        _encode_body,
        out_type=jax.ShapeDtypeStruct((N_LEVELS, 2 * CPTS), jnp.float32),
        mesh=plsc.VectorSubcoreMesh(core_axis_name="c", subcore_axis_name="s"),
        compiler_params=pltpu.CompilerParams(needs_layout_passes=False),
        scratch_types=[
            pltpu.VMEM((2 * B,), jnp.int32),
            pltpu.VMEM((2 * B,), jnp.int32),
            pltpu.VMEM((2 * B,), jnp.float32),
            pltpu.VMEM((2 * B,), jnp.float32),
            pltpu.VMEM((GL,), jnp.int32),
            pltpu.VMEM((GL,), jnp.int32),
            pltpu.VMEM((NSL, 2, B), jnp.float32),
            pltpu.VMEM((NSL, 2, B), jnp.float32),
            pltpu.VMEM((GL,), jnp.float32),
            pltpu.VMEM((GL,), jnp.float32),
            pltpu.VMEM((N_LEVELS, B), jnp.float32),
            pltpu.VMEM((N_LEVELS, B), jnp.float32),
            pltpu.VMEM((TS,), jnp.float32),
            pltpu.SemaphoreType.DMA,
            pltpu.SemaphoreType.DMA,
            pltpu.SemaphoreType.DMA,
            pltpu.SemaphoreType.DMA,
            pltpu.SemaphoreType.DMA,
        ],
    )


_hash_encode = [_make_encode(h * CPTS) for h in range(NCHUNK)]


BT = 4096


def _mlp_body(e_ref, a0, a1, b0, w1, b1, w2, b2, w3, b3, o_ref):
    e = e_ref[...]
    er = jnp.concatenate([e[:, 1:], e[:, :1]], axis=1)
    h = jnp.dot(a0[...], e, preferred_element_type=jnp.float32)
    h = h + jnp.dot(a1[...], er, preferred_element_type=jnp.float32)
    h = jnp.sin(FIRST_OMEGA * (h + b0[...]))
    h = jnp.sin(jnp.dot(w1[...], h, preferred_element_type=jnp.float32) + b1[...])
    h = jnp.sin(jnp.dot(w2[...], h, preferred_element_type=jnp.float32) + b2[...])
    o_ref[...] = jnp.dot(w3[...], h, preferred_element_type=jnp.float32) + b3[...]


def _mlp(eI, A0, A1, b0, W1, b1, W2, b2, W3, b3):
    full = lambda shape: pl.BlockSpec(shape, lambda i: (0, 0))
    return pl.pallas_call(
        _mlp_body,
        grid=(CPTS // BT,),
        in_specs=[
            pl.BlockSpec((N_LEVELS, 2 * BT), lambda i: (0, i)),
            full((HIDDEN, N_LEVELS)), full((HIDDEN, N_LEVELS)),
            full((HIDDEN, 1)),
            full((HIDDEN, HIDDEN)), full((HIDDEN, 1)),
            full((HIDDEN, HIDDEN)), full((HIDDEN, 1)),
            full((1, HIDDEN)), full((1, 1)),
        ],
        out_specs=pl.BlockSpec((1, 2 * BT), lambda i: (0, i)),
        out_shape=jax.ShapeDtypeStruct((1, 2 * CPTS), jnp.float32),
    )(eI, A0, A1, b0, W1, b1, W2, b2, W3, b3)


def kernel(input, table, W0, b0, W1, b1, W2, b2, W3, b3):
    xy = input.T.reshape(2 * N_PTS)
    ftab = table.transpose(0, 2, 1).reshape(N_LEVELS * FPL * T)
    A0 = W0[:, 0::2]
    A1 = W0[:, 1::2]
    mlp_args = (A0, A1, b0.reshape(HIDDEN, 1), W1, b1.reshape(HIDDEN, 1),
                W2, b2.reshape(HIDDEN, 1), W3, b3.reshape(1, 1))
    outs = []
    for h in range(NCHUNK):
        eI = _hash_encode[h](xy, ftab)
        outs.append(_mlp(eI, *mlp_args))
    return jnp.concatenate(
        [o.reshape(2 * CPTS)[0::2] for o in outs]).reshape(N_PTS, 1)
